# baseline (speedup 1.0000x reference)
import jax
import jax.numpy as jnp
from jax import lax
from jax.experimental import pallas as pl
from jax.experimental.pallas import tpu as pltpu
from jax.experimental.pallas import tpu_sc as plsc

_B = 16384
_D = 64
_L = 16
_NC = 2
_NS = 16
_SHIFT = 15
_WSZ = 1 << _SHIFT
_CH = 128
_NCHUNK = _B // _CH

_SENT = 0x7FFFFFFF


def _shift_up(x):
    i = jnp.minimum(lax.iota(jnp.int32, _L) + 1, _L - 1)
    dnums = lax.GatherDimensionNumbers(
        offset_dims=(), collapsed_slice_dims=(0,), start_index_map=(0,))
    return lax.gather(x, i[:, None], dnums, (1,),
                      mode=lax.GatherScatterMode.PROMISE_IN_BOUNDS)


def _body(idx_hbm, val_hbm, lab_hbm, out_hbm, outlab_hbm,
          idx_v, W, my_i, my_loc, rows, lrow, sem):
    wid = lax.axis_index("s") * _NC + lax.axis_index("c")
    with jax.named_scope("stage_idx"):
        pltpu.sync_copy(idx_hbm, idx_v)
    lane = lax.iota(jnp.int32, _L)

    def p1(k, cnt):
        v = idx_v[pl.ds(k * _L, _L)]
        owner = lax.shift_right_logical(v, _SHIFT)
        m = owner == wid
        local = lax.bitwise_and(v, _WSZ - 1)
        key = jnp.where(m, lax.bitwise_or(lax.shift_left(local, 4), lane),
                        _SENT)
        skey, _ = plsc.sort_key_val(key, key)
        sloc_cmp = lax.shift_right_logical(skey, 4)
        keep = (sloc_cmp != _shift_up(sloc_cmp)) | (lane == _L - 1)
        wr = keep & (skey != _SENT)
        sloc = lax.bitwise_and(sloc_cmp, _WSZ - 1)
        j_sorted = k * _L + lax.bitwise_and(skey, _L - 1)
        plsc.store_scatter(W, [sloc], j_sorted, mask=wr)
        cum = plsc.cumsum(m.astype(jnp.int32))
        addr = cnt + cum - 1
        hi = lax.shift_right_logical(addr, 7)
        lo = lax.bitwise_and(addr, _CH - 1)
        plsc.store_scatter(my_i, [hi, lo], k * _L + lane, mask=m)
        plsc.store_scatter(my_loc, [hi, lo], local, mask=m)
        return cnt + jnp.sum(m.astype(jnp.int32))

    with jax.named_scope("p1_scan"):
        cnt = lax.fori_loop(0, _B // _L, p1, jnp.int32(0))

    nchunks = lax.shift_right_logical(cnt + _CH - 1, 7)
    zero16 = jnp.zeros((_L,), jnp.int32)
    i0 = plsc.load_gather(my_i, [zero16, zero16])
    l0 = lax.bitwise_and(plsc.load_gather(my_loc, [zero16, zero16]), _WSZ - 1)
    r0 = plsc.load_gather(W, [l0])

    def p2a(t, _):
        pos = t * _L + lane
        hi = lax.shift_right_logical(pos, 7)
        lo = lax.bitwise_and(pos, _CH - 1)
        valid = pos < cnt
        loc = lax.bitwise_and(plsc.load_gather(my_loc, [hi, lo]), _WSZ - 1)
        r = jnp.where(valid, plsc.load_gather(W, [loc]), r0)
        iv = jnp.where(valid, plsc.load_gather(my_i, [hi, lo]), i0)
        plsc.store_scatter(my_loc, [hi, lo], r)
        plsc.store_scatter(my_i, [hi, lo], iv)
        return 0

    with jax.named_scope("p2a_resolve"):
        lax.fori_loop(0, nchunks * (_CH // _L), p2a, 0)

    def p2b(c, _):
        r_row = my_loc.at[c]
        i_row = my_i.at[c]
        pltpu.async_copy(val_hbm.at[r_row], rows, sem).wait()
        pltpu.async_copy(rows, out_hbm.at[i_row], sem).wait()
        pltpu.async_copy(lab_hbm.at[r_row], lrow, sem).wait()
        pltpu.async_copy(lrow, outlab_hbm.at[i_row], sem).wait()
        return 0

    with jax.named_scope("p2b_dma"):
        lax.fori_loop(0, nchunks, p2b, 0)


def kernel(idx, val, labels, buffer_imgs, buffer_labels):
    del buffer_imgs, buffer_labels
    f = pl.kernel(
        _body,
        out_type=(
            jax.ShapeDtypeStruct((_B, _D), jnp.float32),
            jax.ShapeDtypeStruct((_B,), jnp.int32),
        ),
        mesh=plsc.VectorSubcoreMesh(core_axis_name="c", subcore_axis_name="s"),
        compiler_params=pltpu.CompilerParams(
            needs_layout_passes=False, use_tc_tiling_on_sc=False),
        scratch_types=[
            pltpu.VMEM((_B,), jnp.int32),
            pltpu.VMEM((_WSZ,), jnp.int32),
            pltpu.VMEM((_NCHUNK, _CH), jnp.int32),
            pltpu.VMEM((_NCHUNK, _CH), jnp.int32),
            pltpu.VMEM((_CH, _D), jnp.float32),
            pltpu.VMEM((_CH,), jnp.int32),
            pltpu.SemaphoreType.DMA,
        ],
    )
    return f(idx.astype(jnp.int32), val, labels.astype(jnp.int32))


# speedup vs baseline: 2.2240x; 2.2240x over previous
import jax
import jax.numpy as jnp
from jax import lax
from jax.experimental import pallas as pl
from jax.experimental.pallas import tpu as pltpu
from jax.experimental.pallas import tpu_sc as plsc

_B = 16384
_D = 64
_L = 16
_NC = 2
_NS = 16
_SHIFT = 15
_WSZ = 1 << _SHIFT
_CH = 128
_NCHUNK = _B // _CH

_SENT = 0x7FFFFFFF


def _shift_up(x):
    i = jnp.minimum(lax.iota(jnp.int32, _L) + 1, _L - 1)
    dnums = lax.GatherDimensionNumbers(
        offset_dims=(), collapsed_slice_dims=(0,), start_index_map=(0,))
    return lax.gather(x, i[:, None], dnums, (1,),
                      mode=lax.GatherScatterMode.PROMISE_IN_BOUNDS)


def _body(idx_hbm, val_hbm, lab_hbm, out_hbm, outlab_hbm,
          idx_v, lab_v, W, my_i, my_loc, rows, lrow, sem):
    wid = lax.axis_index("s") * _NC + lax.axis_index("c")
    with jax.named_scope("stage_idx"):
        pltpu.sync_copy(idx_hbm, idx_v)
        pltpu.sync_copy(lab_hbm, lab_v)
    lane = lax.iota(jnp.int32, _L)

    def p1(k, cnt):
        v = idx_v[pl.ds(k * _L, _L)]
        owner = lax.shift_right_logical(v, _SHIFT)
        m = owner == wid
        local = lax.bitwise_and(v, _WSZ - 1)
        key = jnp.where(m, lax.bitwise_or(lax.shift_left(local, 4), lane),
                        _SENT)
        skey, _ = plsc.sort_key_val(key, key)
        sloc_cmp = lax.shift_right_logical(skey, 4)
        keep = (sloc_cmp != _shift_up(sloc_cmp)) | (lane == _L - 1)
        wr = keep & (skey != _SENT)
        sloc = lax.bitwise_and(sloc_cmp, _WSZ - 1)
        j_sorted = k * _L + lax.bitwise_and(skey, _L - 1)
        plsc.store_scatter(W, [sloc], j_sorted, mask=wr)
        cum = plsc.cumsum(m.astype(jnp.int32))
        addr = cnt + cum - 1
        hi = lax.shift_right_logical(addr, 7)
        lo = lax.bitwise_and(addr, _CH - 1)
        plsc.store_scatter(my_i, [hi, lo], k * _L + lane, mask=m)
        plsc.store_scatter(my_loc, [hi, lo], local, mask=m)
        return cnt + jnp.sum(m.astype(jnp.int32))

    with jax.named_scope("p1_scan"):
        cnt = lax.fori_loop(0, _B // _L, p1, jnp.int32(0))

    nchunks = lax.shift_right_logical(cnt + _CH - 1, 7)
    zero16 = jnp.zeros((_L,), jnp.int32)
    i0 = plsc.load_gather(my_i, [zero16, zero16])
    l0 = lax.bitwise_and(plsc.load_gather(my_loc, [zero16, zero16]), _WSZ - 1)
    r0 = plsc.load_gather(W, [l0])

    def p2a(t, _):
        pos = t * _L + lane
        hi = lax.shift_right_logical(pos, 7)
        lo = lax.bitwise_and(pos, _CH - 1)
        valid = pos < cnt
        loc = lax.bitwise_and(plsc.load_gather(my_loc, [hi, lo]), _WSZ - 1)
        r = jnp.where(valid, plsc.load_gather(W, [loc]), r0)
        iv = jnp.where(valid, plsc.load_gather(my_i, [hi, lo]), i0)
        plsc.store_scatter(my_loc, [hi, lo], r)
        plsc.store_scatter(my_i, [hi, lo], iv)
        return 0

    with jax.named_scope("p2a_resolve"):
        lax.fori_loop(0, nchunks * (_CH // _L), p2a, 0)

    def p2b(c, _):
        r_row = my_loc.at[c]
        i_row = my_i.at[c]
        cp = pltpu.async_copy(val_hbm.at[r_row], rows, sem)
        for s in range(_CH // _L):
            p = s * _L + lane
            r = plsc.load_gather(my_loc, [jnp.broadcast_to(c, (_L,)), p])
            lab = plsc.load_gather(lab_v, [r])
            plsc.store_scatter(lrow, [p, jnp.zeros((_L,), jnp.int32)], lab)
        cp.wait()
        pltpu.async_copy(rows, out_hbm.at[i_row], sem).wait()
        pltpu.async_copy(lrow, outlab_hbm.at[i_row], sem).wait()
        return 0

    with jax.named_scope("p2b_dma"):
        lax.fori_loop(0, nchunks, p2b, 0)


def kernel(idx, val, labels, buffer_imgs, buffer_labels):
    del buffer_imgs, buffer_labels
    f = pl.kernel(
        _body,
        out_type=(
            jax.ShapeDtypeStruct((_B, _D), jnp.float32),
            jax.ShapeDtypeStruct((_B, _L), jnp.int32),
        ),
        mesh=plsc.VectorSubcoreMesh(core_axis_name="c", subcore_axis_name="s"),
        compiler_params=pltpu.CompilerParams(
            needs_layout_passes=False, use_tc_tiling_on_sc=False),
        scratch_types=[
            pltpu.VMEM((_B,), jnp.int32),
            pltpu.VMEM((_B,), jnp.int32),
            pltpu.VMEM((_WSZ,), jnp.int32),
            pltpu.VMEM((_NCHUNK, _CH), jnp.int32),
            pltpu.VMEM((_NCHUNK, _CH), jnp.int32),
            pltpu.VMEM((_CH, _D), jnp.float32),
            pltpu.VMEM((_CH, _L), jnp.int32),
            pltpu.SemaphoreType.DMA,
        ],
    )
    out_imgs, out_lab_pad = f(idx.astype(jnp.int32), val,
                              labels.astype(jnp.int32))
    return out_imgs, out_lab_pad[:, 0]


# speedup vs baseline: 2.3424x; 1.0532x over previous
import jax
import jax.numpy as jnp
from jax import lax
from jax.experimental import pallas as pl
from jax.experimental.pallas import tpu as pltpu
from jax.experimental.pallas import tpu_sc as plsc

_B = 16384
_D = 64
_L = 16
_NC = 2
_NS = 16
_SHIFT = 15
_WSZ = 1 << _SHIFT
_CH = 128
_NCHUNK = _B // _CH

_SENT = 0x7FFFFFFF


def _shift_up(x):
    i = jnp.minimum(lax.iota(jnp.int32, _L) + 1, _L - 1)
    dnums = lax.GatherDimensionNumbers(
        offset_dims=(), collapsed_slice_dims=(0,), start_index_map=(0,))
    return lax.gather(x, i[:, None], dnums, (1,),
                      mode=lax.GatherScatterMode.PROMISE_IN_BOUNDS)


def _body(idx_hbm, val_hbm, lab_hbm, out_hbm, outlab_hbm,
          idx_v, lab_v, W, my_i, my_loc, rows2, lrow2,
          gsem, ssem, lsem, isem):
    wid = lax.axis_index("s") * _NC + lax.axis_index("c")
    lane = lax.iota(jnp.int32, _L)

    with jax.named_scope("stage_idx"):
        cp_idx = pltpu.async_copy(idx_hbm, idx_v, isem)
        cp_lab = pltpu.async_copy(lab_hbm, lab_v, lsem)
        cp_idx.wait()

    def p1(k, cnt_vec):
        v = idx_v[pl.ds(k * _L, _L)]
        owner = lax.shift_right_logical(v, _SHIFT)
        m = owner == wid
        local = lax.bitwise_and(v, _WSZ - 1)
        key = jnp.where(m, lax.bitwise_or(lax.shift_left(local, 4), lane),
                        _SENT)
        skey, _ = plsc.sort_key_val(key, key)
        sloc_cmp = lax.shift_right_logical(skey, 4)
        keep = (sloc_cmp != _shift_up(sloc_cmp)) | (lane == _L - 1)
        sm = skey != _SENT
        wr = keep & sm
        sloc = lax.bitwise_and(sloc_cmp, _WSZ - 1)
        j_sorted = k * _L + lax.bitwise_and(skey, _L - 1)
        plsc.store_scatter(W, [sloc], j_sorted, mask=wr)
        addr = cnt_vec + lane
        hi = lax.shift_right_logical(addr, 7)
        lo = lax.bitwise_and(addr, _CH - 1)
        plsc.store_scatter(my_i, [hi, lo], j_sorted, mask=sm)
        plsc.store_scatter(my_loc, [hi, lo], sloc, mask=sm)
        return cnt_vec + plsc.all_reduce_population_count(m)

    with jax.named_scope("p1_scan"):
        cnt_vec = lax.fori_loop(0, _B // _L, p1,
                                jnp.zeros((_L,), jnp.int32))
        cnt = jnp.max(cnt_vec)

    nchunks = lax.shift_right_logical(cnt + _CH - 1, 7)
    zero16 = jnp.zeros((_L,), jnp.int32)
    i0 = plsc.load_gather(my_i, [zero16, zero16])
    l0 = lax.bitwise_and(plsc.load_gather(my_loc, [zero16, zero16]), _WSZ - 1)
    r0 = plsc.load_gather(W, [l0])

    def p2a(t, _):
        pos = t * _L + lane
        hi = lax.shift_right_logical(pos, 7)
        lo = lax.bitwise_and(pos, _CH - 1)
        valid = pos < cnt
        loc = lax.bitwise_and(plsc.load_gather(my_loc, [hi, lo]), _WSZ - 1)
        r = jnp.where(valid, plsc.load_gather(W, [loc]), r0)
        iv = jnp.where(valid, plsc.load_gather(my_i, [hi, lo]), i0)
        plsc.store_scatter(my_loc, [hi, lo], r)
        plsc.store_scatter(my_i, [hi, lo], iv)
        return 0

    with jax.named_scope("p2a_resolve"):
        lax.fori_loop(0, nchunks * (_CH // _L), p2a, 0)
        cp_lab.wait()

    def _wait_gather():
        pltpu.make_async_copy(
            val_hbm.at[pl.ds(0, _CH)], rows2.at[0], gsem).wait()

    def _wait_scatters():
        pltpu.make_async_copy(
            rows2.at[0], out_hbm.at[pl.ds(0, _CH)], ssem).wait()
        pltpu.make_async_copy(
            lrow2.at[0], outlab_hbm.at[pl.ds(0, _CH)], lsem).wait()

    def p2b(c, _):
        buf = lax.bitwise_and(c, 1)
        nbuf = lax.bitwise_and(c + 1, 1)

        @pl.when(c >= 1)
        def _():
            _wait_scatters()

        @pl.when(c + 1 < nchunks)
        def _():
            pltpu.async_copy(val_hbm.at[my_loc.at[c + 1]], rows2.at[nbuf],
                             gsem)

        for s in range(_CH // _L):
            p = s * _L + lane
            r = my_loc[c, pl.ds(s * _L, _L)]
            lab = plsc.load_gather(lab_v, [r])
            plsc.store_scatter(lrow2.at[buf], [p, zero16], lab)
        _wait_gather()
        pltpu.async_copy(rows2.at[buf], out_hbm.at[my_i.at[c]], ssem)
        pltpu.async_copy(lrow2.at[buf], outlab_hbm.at[my_i.at[c]], lsem)
        return 0

    with jax.named_scope("p2b_dma"):
        @pl.when(nchunks > 0)
        def _():
            pltpu.async_copy(val_hbm.at[my_loc.at[0]], rows2.at[0], gsem)

        lax.fori_loop(0, nchunks, p2b, 0)

        @pl.when(nchunks > 0)
        def _():
            _wait_scatters()


def kernel(idx, val, labels, buffer_imgs, buffer_labels):
    del buffer_imgs, buffer_labels
    f = pl.kernel(
        _body,
        out_type=(
            jax.ShapeDtypeStruct((_B, _D), jnp.float32),
            jax.ShapeDtypeStruct((_B, _L), jnp.int32),
        ),
        mesh=plsc.VectorSubcoreMesh(core_axis_name="c", subcore_axis_name="s"),
        compiler_params=pltpu.CompilerParams(
            needs_layout_passes=False, use_tc_tiling_on_sc=False),
        scratch_types=[
            pltpu.VMEM((_B,), jnp.int32),
            pltpu.VMEM((_B,), jnp.int32),
            pltpu.VMEM((_WSZ,), jnp.int32),
            pltpu.VMEM((_NCHUNK, _CH), jnp.int32),
            pltpu.VMEM((_NCHUNK, _CH), jnp.int32),
            pltpu.VMEM((2, _CH, _D), jnp.float32),
            pltpu.VMEM((2, _CH, _L), jnp.int32),
            pltpu.SemaphoreType.DMA,
            pltpu.SemaphoreType.DMA,
            pltpu.SemaphoreType.DMA,
            pltpu.SemaphoreType.DMA,
        ],
    )
    out_imgs, out_lab_pad = f(idx.astype(jnp.int32), val,
                              labels.astype(jnp.int32))
    return out_imgs, out_lab_pad[:, 0]


# speedup vs baseline: 2.8663x; 1.2237x over previous
import jax
import jax.numpy as jnp
from jax import lax
from jax.experimental import pallas as pl
from jax.experimental.pallas import tpu as pltpu
from jax.experimental.pallas import tpu_sc as plsc

_B = 16384
_D = 64
_L = 16
_NC = 2
_NS = 16
_SHIFT = 15
_WSZ = 1 << _SHIFT
_CH = 128
_NCHUNK = _B // _CH

_SENT = 0x7FFFFFFF


def _shift_up(x):
    i = jnp.minimum(lax.iota(jnp.int32, _L) + 1, _L - 1)
    dnums = lax.GatherDimensionNumbers(
        offset_dims=(), collapsed_slice_dims=(0,), start_index_map=(0,))
    return lax.gather(x, i[:, None], dnums, (1,),
                      mode=lax.GatherScatterMode.PROMISE_IN_BOUNDS)


def _body(idx_hbm, val_hbm, lab_hbm, out_hbm, outlab_hbm,
          idx_v, lab_v, W, my_i, rlist, rows2, lrow2,
          gsem, ssem, lsem, isem):
    wid = lax.axis_index("s") * _NC + lax.axis_index("c")
    lane = lax.iota(jnp.int32, _L)

    with jax.named_scope("stage_idx"):
        cp_idx = pltpu.async_copy(idx_hbm, idx_v, isem)
        cp_lab = pltpu.async_copy(lab_hbm, lab_v, lsem)
        cp_idx.wait()

    _U = 4

    def p1(kk, cnt_vec):
        sorted_keys = []
        masks = []
        for u in range(_U):
            k = kk * _U + u
            v = idx_v[pl.ds(k * _L, _L)]
            owner = lax.shift_right_logical(v, _SHIFT)
            m = owner == wid
            local = lax.bitwise_and(v, _WSZ - 1)
            key = jnp.where(m, lax.bitwise_or(lax.shift_left(local, 4), lane),
                            _SENT)
            skey, _ = plsc.sort_key_val(key, key)
            sorted_keys.append(skey)
            masks.append(m)
        for u in range(_U):
            k = kk * _U + u
            skey = sorted_keys[u]
            sloc_cmp = lax.shift_right_logical(skey, 4)
            keep = (sloc_cmp != _shift_up(sloc_cmp)) | (lane == _L - 1)
            sm = skey != _SENT
            wr = keep & sm
            sloc = lax.bitwise_and(sloc_cmp, _WSZ - 1)
            j_sorted = k * _L + lax.bitwise_and(skey, _L - 1)
            plsc.store_scatter(W, [sloc], j_sorted, mask=wr)
            addr = cnt_vec + lane
            hi = lax.shift_right_logical(addr, 7)
            lo = lax.bitwise_and(addr, _CH - 1)
            plsc.store_scatter(my_i, [hi, lo], j_sorted, mask=sm)
            cnt_vec = cnt_vec + plsc.all_reduce_population_count(masks[u])
        return cnt_vec

    with jax.named_scope("p1_scan"):
        cnt_vec = lax.fori_loop(0, _B // (_L * _U), p1,
                                jnp.zeros((_L,), jnp.int32))
        cnt = jnp.max(cnt_vec)

    nchunks = lax.shift_right_logical(cnt + _CH - 1, 7)
    zero16 = jnp.zeros((_L,), jnp.int32)
    i0 = plsc.load_gather(my_i, [zero16, zero16])
    l0 = lax.bitwise_and(
        plsc.load_gather(idx_v, [lax.bitwise_and(i0, _B - 1)]), _WSZ - 1)
    r0 = plsc.load_gather(W, [l0])

    def p2a(t, _):
        pos = t * _L + lane
        hi = lax.shift_right_logical(pos, 7)
        lo = lax.bitwise_and(pos, _CH - 1)
        valid = pos < cnt
        iv_raw = plsc.load_gather(my_i, [hi, lo])
        loc = lax.bitwise_and(
            plsc.load_gather(idx_v, [lax.bitwise_and(iv_raw, _B - 1)]),
            _WSZ - 1)
        r = jnp.where(valid, plsc.load_gather(W, [loc]), r0)
        iv = jnp.where(valid, iv_raw, i0)
        plsc.store_scatter(rlist, [hi, lo], r)
        plsc.store_scatter(my_i, [hi, lo], iv)
        return 0

    with jax.named_scope("p2a_resolve"):
        lax.fori_loop(0, nchunks * (_CH // _L), p2a, 0)
        cp_lab.wait()

    def _wait_gather():
        pltpu.make_async_copy(
            val_hbm.at[pl.ds(0, _CH)], rows2.at[0], gsem).wait()

    def _wait_scatters():
        pltpu.make_async_copy(
            rows2.at[0], out_hbm.at[pl.ds(0, _CH)], ssem).wait()
        pltpu.make_async_copy(
            lrow2.at[0], outlab_hbm.at[pl.ds(0, _CH)], lsem).wait()

    def p2b(c, _):
        buf = lax.bitwise_and(c, 1)
        nbuf = lax.bitwise_and(c + 1, 1)

        @pl.when(c >= 1)
        def _():
            _wait_scatters()

        @pl.when(c + 1 < nchunks)
        def _():
            pltpu.async_copy(val_hbm.at[rlist.at[c + 1]], rows2.at[nbuf],
                             gsem)

        for s in range(_CH // _L):
            p = s * _L + lane
            r = rlist[c, pl.ds(s * _L, _L)]
            lab = plsc.load_gather(lab_v, [r])
            plsc.store_scatter(lrow2.at[buf], [p, zero16], lab)
        _wait_gather()
        pltpu.async_copy(rows2.at[buf], out_hbm.at[my_i.at[c]], ssem)
        pltpu.async_copy(lrow2.at[buf], outlab_hbm.at[my_i.at[c]], lsem)
        return 0

    with jax.named_scope("p2b_dma"):
        @pl.when(nchunks > 0)
        def _():
            pltpu.async_copy(val_hbm.at[rlist.at[0]], rows2.at[0], gsem)

        lax.fori_loop(0, nchunks, p2b, 0)

        @pl.when(nchunks > 0)
        def _():
            _wait_scatters()


def kernel(idx, val, labels, buffer_imgs, buffer_labels):
    del buffer_imgs, buffer_labels
    f = pl.kernel(
        _body,
        out_type=(
            jax.ShapeDtypeStruct((_B, _D), jnp.float32),
            jax.ShapeDtypeStruct((_B, _L), jnp.int32),
        ),
        mesh=plsc.VectorSubcoreMesh(core_axis_name="c", subcore_axis_name="s"),
        compiler_params=pltpu.CompilerParams(
            needs_layout_passes=False, use_tc_tiling_on_sc=False),
        scratch_types=[
            pltpu.VMEM((_B,), jnp.int32),
            pltpu.VMEM((_B,), jnp.int32),
            pltpu.VMEM((_WSZ,), jnp.int32),
            pltpu.VMEM((_NCHUNK, _CH), jnp.int32),
            pltpu.VMEM((_NCHUNK, _CH), jnp.int32),
            pltpu.VMEM((2, _CH, _D), jnp.float32),
            pltpu.VMEM((2, _CH, _L), jnp.int32),
            pltpu.SemaphoreType.DMA,
            pltpu.SemaphoreType.DMA,
            pltpu.SemaphoreType.DMA,
            pltpu.SemaphoreType.DMA,
        ],
    )
    out_imgs, out_lab_pad = f(idx.astype(jnp.int32), val,
                              labels.astype(jnp.int32))
    return out_imgs, out_lab_pad[:, 0]


# speedup vs baseline: 3.2236x; 1.1247x over previous
import jax
import jax.numpy as jnp
from jax import lax
from jax.experimental import pallas as pl
from jax.experimental.pallas import tpu as pltpu
from jax.experimental.pallas import tpu_sc as plsc

_B = 16384
_D = 64
_L = 16
_NC = 2
_NS = 16
_SHIFT = 15
_WSZ = 1 << _SHIFT
_CH = 128
_NCHUNK = _B // _CH

_SENT = 0x7FFFFFFF


def _shift_up(x):
    i = jnp.minimum(lax.iota(jnp.int32, _L) + 1, _L - 1)
    dnums = lax.GatherDimensionNumbers(
        offset_dims=(), collapsed_slice_dims=(0,), start_index_map=(0,))
    return lax.gather(x, i[:, None], dnums, (1,),
                      mode=lax.GatherScatterMode.PROMISE_IN_BOUNDS)


def _body(idx_hbm, val_hbm, lab_hbm, out_hbm, outlab_hbm,
          idx_v, lab_v, W, my_i, rlist, rows2, labbuf, zbuf, shared_lab,
          gsem, ssem, lsem, isem):
    cid = lax.axis_index("c")
    sid = lax.axis_index("s")
    wid = sid * _NC + cid
    lane = lax.iota(jnp.int32, _L)
    _SL = _B // _NS

    with jax.named_scope("stage_idx"):
        cp_idx = pltpu.async_copy(idx_hbm, idx_v, isem)
        cp_lab = pltpu.async_copy(lab_hbm, lab_v, lsem)
        for s in range(_SL // _L):
            zbuf[pl.ds(s * _L, _L)] = jnp.zeros((_L,), jnp.int32)
        pltpu.sync_copy(zbuf, shared_lab.at[pl.ds(sid * _SL, _SL)])
        plsc.subcore_barrier()
        cp_idx.wait()

    _U = 4

    def p1(kk, cnt_vec):
        sorted_keys = []
        masks = []
        for u in range(_U):
            k = kk * _U + u
            v = idx_v[pl.ds(k * _L, _L)]
            owner = lax.shift_right_logical(v, _SHIFT)
            m = owner == wid
            local = lax.bitwise_and(v, _WSZ - 1)
            key = jnp.where(m, lax.bitwise_or(lax.shift_left(local, 4), lane),
                            _SENT)
            skey, _ = plsc.sort_key_val(key, key)
            sorted_keys.append(skey)
            masks.append(m)
        for u in range(_U):
            k = kk * _U + u
            skey = sorted_keys[u]
            sloc_cmp = lax.shift_right_logical(skey, 4)
            keep = (sloc_cmp != _shift_up(sloc_cmp)) | (lane == _L - 1)
            sm = skey != _SENT
            wr = keep & sm
            sloc = lax.bitwise_and(sloc_cmp, _WSZ - 1)
            j_sorted = k * _L + lax.bitwise_and(skey, _L - 1)
            plsc.store_scatter(W, [sloc], j_sorted, mask=wr)
            addr = cnt_vec + lane
            hi = lax.shift_right_logical(addr, 7)
            lo = lax.bitwise_and(addr, _CH - 1)
            plsc.store_scatter(my_i, [hi, lo], j_sorted, mask=sm)
            cnt_vec = cnt_vec + plsc.all_reduce_population_count(masks[u])
        return cnt_vec

    with jax.named_scope("p1_scan"):
        cnt_vec = lax.fori_loop(0, _B // (_L * _U), p1,
                                jnp.zeros((_L,), jnp.int32))
        cnt = jnp.max(cnt_vec)

    nchunks = lax.shift_right_logical(cnt + _CH - 1, 7)
    zero16 = jnp.zeros((_L,), jnp.int32)
    i0 = plsc.load_gather(my_i, [zero16, zero16])
    l0 = lax.bitwise_and(
        plsc.load_gather(idx_v, [lax.bitwise_and(i0, _B - 1)]), _WSZ - 1)
    r0 = plsc.load_gather(W, [l0])

    def p2a(t, _):
        pos = t * _L + lane
        hi = lax.shift_right_logical(pos, 7)
        lo = lax.bitwise_and(pos, _CH - 1)
        valid = pos < cnt
        iv_raw = plsc.load_gather(my_i, [hi, lo])
        loc = lax.bitwise_and(
            plsc.load_gather(idx_v, [lax.bitwise_and(iv_raw, _B - 1)]),
            _WSZ - 1)
        r = jnp.where(valid, plsc.load_gather(W, [loc]), r0)
        iv = jnp.where(valid, iv_raw, i0)
        plsc.store_scatter(rlist, [hi, lo], r)
        plsc.store_scatter(my_i, [hi, lo], iv)
        return 0

    with jax.named_scope("p2a_resolve"):
        lax.fori_loop(0, nchunks * (_CH // _L), p2a, 0)
        cp_lab.wait()

    def _wait_gather():
        pltpu.make_async_copy(
            val_hbm.at[pl.ds(0, _CH)], rows2.at[0], gsem).wait()

    def _wait_scatters():
        pltpu.make_async_copy(
            rows2.at[0], out_hbm.at[pl.ds(0, _CH)], ssem).wait()
        pltpu.make_async_copy(
            labbuf.at[0], shared_lab.at[pl.ds(0, _CH)], lsem).wait()

    def p2b(c, _):
        buf = lax.bitwise_and(c, 1)
        nbuf = lax.bitwise_and(c + 1, 1)

        @pl.when(c >= 1)
        def _():
            _wait_scatters()

        @pl.when(c + 1 < nchunks)
        def _():
            pltpu.async_copy(val_hbm.at[rlist.at[c + 1]], rows2.at[nbuf],
                             gsem)

        for s in range(_CH // _L):
            p = s * _L + lane
            r = rlist[c, pl.ds(s * _L, _L)]
            lab = plsc.load_gather(lab_v, [r])
            lab = jnp.where(c * _CH + p < cnt, lab, 0)
            plsc.store_scatter(labbuf.at[buf], [p], lab)
        _wait_gather()
        pltpu.async_copy(rows2.at[buf], out_hbm.at[my_i.at[c]], ssem)
        pltpu.async_copy(labbuf.at[buf], shared_lab.at[my_i.at[c]], lsem,
                         add=True)
        return 0

    with jax.named_scope("p2b_dma"):
        @pl.when(nchunks > 0)
        def _():
            pltpu.async_copy(val_hbm.at[rlist.at[0]], rows2.at[0], gsem)

        lax.fori_loop(0, nchunks, p2b, 0)

        @pl.when(nchunks > 0)
        def _():
            _wait_scatters()

    with jax.named_scope("p2c_labels_out"):
        plsc.subcore_barrier()
        pltpu.sync_copy(shared_lab.at[pl.ds(sid * _SL, _SL)],
                        outlab_hbm.at[cid, pl.ds(sid * _SL, _SL)])


def kernel(idx, val, labels, buffer_imgs, buffer_labels):
    del buffer_imgs, buffer_labels
    f = pl.kernel(
        _body,
        out_type=(
            jax.ShapeDtypeStruct((_B, _D), jnp.float32),
            jax.ShapeDtypeStruct((_NC, _B), jnp.int32),
        ),
        mesh=plsc.VectorSubcoreMesh(core_axis_name="c", subcore_axis_name="s"),
        compiler_params=pltpu.CompilerParams(
            needs_layout_passes=False, use_tc_tiling_on_sc=False),
        scratch_types=[
            pltpu.VMEM((_B,), jnp.int32),
            pltpu.VMEM((_B,), jnp.int32),
            pltpu.VMEM((_WSZ,), jnp.int32),
            pltpu.VMEM((_NCHUNK, _CH), jnp.int32),
            pltpu.VMEM((_NCHUNK, _CH), jnp.int32),
            pltpu.VMEM((2, _CH, _D), jnp.float32),
            pltpu.VMEM((2, _CH), jnp.int32),
            pltpu.VMEM((_B // _NS,), jnp.int32),
            pltpu.VMEM_SHARED((_B,), jnp.int32),
            pltpu.SemaphoreType.DMA,
            pltpu.SemaphoreType.DMA,
            pltpu.SemaphoreType.DMA,
            pltpu.SemaphoreType.DMA,
        ],
    )
    out_imgs, lab_parts = f(idx.astype(jnp.int32), val,
                            labels.astype(jnp.int32))
    return out_imgs, lab_parts[0] + lab_parts[1]


# speedup vs baseline: 3.5814x; 1.1110x over previous
import jax
import jax.numpy as jnp
from jax import lax
from jax.experimental import pallas as pl
from jax.experimental.pallas import tpu as pltpu
from jax.experimental.pallas import tpu_sc as plsc

_B = 16384
_D = 64
_L = 16
_NC = 2
_NS = 16
_SHIFT = 15
_WSZ = 1 << _SHIFT
_DP = 128
_CH = 64
_CHS = 6
_NCHUNK = _B // _CH

_SENT = 0x7FFFFFFF


def _shift_up(x):
    i = jnp.minimum(lax.iota(jnp.int32, _L) + 1, _L - 1)
    dnums = lax.GatherDimensionNumbers(
        offset_dims=(), collapsed_slice_dims=(0,), start_index_map=(0,))
    return lax.gather(x, i[:, None], dnums, (1,),
                      mode=lax.GatherScatterMode.PROMISE_IN_BOUNDS)


def _body(idx_hbm, val_hbm, lab_hbm, out_hbm, lab0_hbm, lab1_hbm,
          idx_v, lab_v, W, my_i, rlist, rows2, labbuf, zbuf, shared_lab,
          gsem, ssem, lsem, isem):
    cid = lax.axis_index("c")
    sid = lax.axis_index("s")
    wid = sid * _NC + cid
    lane = lax.iota(jnp.int32, _L)
    _SL = _B // _NS

    with jax.named_scope("stage_idx"):
        cp_idx = pltpu.async_copy(idx_hbm, idx_v, isem)
        cp_lab = pltpu.async_copy(lab_hbm, lab_v, lsem)
        for s in range(_SL // _L):
            zbuf[pl.ds(s * _L, _L)] = jnp.zeros((_L,), jnp.int32)
        pltpu.sync_copy(zbuf, shared_lab.at[pl.ds(sid * _SL, _SL)])
        plsc.subcore_barrier()
        cp_idx.wait()

    _U = 4

    def p1(kk, cnt_vec):
        sorted_keys = []
        masks = []
        for u in range(_U):
            k = kk * _U + u
            v = idx_v[pl.ds(k * _L, _L)]
            owner = lax.shift_right_logical(v, _SHIFT)
            m = owner == wid
            local = lax.bitwise_and(v, _WSZ - 1)
            key = jnp.where(m, lax.bitwise_or(lax.shift_left(local, 4), lane),
                            _SENT)
            skey, _ = plsc.sort_key_val(key, key)
            sorted_keys.append(skey)
            masks.append(m)
        for u in range(_U):
            k = kk * _U + u
            skey = sorted_keys[u]
            sloc_cmp = lax.shift_right_logical(skey, 4)
            keep = (sloc_cmp != _shift_up(sloc_cmp)) | (lane == _L - 1)
            sm = skey != _SENT
            wr = keep & sm
            sloc = lax.bitwise_and(sloc_cmp, _WSZ - 1)
            j_sorted = k * _L + lax.bitwise_and(skey, _L - 1)
            plsc.store_scatter(W, [sloc], j_sorted, mask=wr)
            addr = cnt_vec + lane
            hi = lax.shift_right_logical(addr, _CHS)
            lo = lax.bitwise_and(addr, _CH - 1)
            plsc.store_scatter(my_i, [hi, lo], j_sorted, mask=sm)
            cnt_vec = cnt_vec + plsc.all_reduce_population_count(masks[u])
        return cnt_vec

    with jax.named_scope("p1_scan"):
        cnt_vec = lax.fori_loop(0, _B // (_L * _U), p1,
                                jnp.zeros((_L,), jnp.int32))
        cnt = jnp.max(cnt_vec)

    nchunks = lax.shift_right_logical(cnt + _CH - 1, _CHS)
    zero16 = jnp.zeros((_L,), jnp.int32)
    i0 = plsc.load_gather(my_i, [zero16, zero16])
    l0 = lax.bitwise_and(
        plsc.load_gather(idx_v, [lax.bitwise_and(i0, _B - 1)]), _WSZ - 1)
    r0 = plsc.load_gather(W, [l0])

    def p2a(t, _):
        pos = t * _L + lane
        hi = lax.shift_right_logical(pos, _CHS)
        lo = lax.bitwise_and(pos, _CH - 1)
        valid = pos < cnt
        iv_raw = plsc.load_gather(my_i, [hi, lo])
        loc = lax.bitwise_and(
            plsc.load_gather(idx_v, [lax.bitwise_and(iv_raw, _B - 1)]),
            _WSZ - 1)
        r = jnp.where(valid, plsc.load_gather(W, [loc]), r0)
        iv = jnp.where(valid, iv_raw, i0)
        plsc.store_scatter(rlist, [hi, lo], r)
        plsc.store_scatter(my_i, [hi, lo], iv)
        return 0

    with jax.named_scope("p2a_resolve"):
        lax.fori_loop(0, nchunks * (_CH // _L), p2a, 0)
        cp_lab.wait()

    def _wait_gather():
        pltpu.make_async_copy(
            val_hbm.at[pl.ds(0, _CH)], rows2.at[0], gsem).wait()

    def _wait_scatters():
        pltpu.make_async_copy(
            rows2.at[0], out_hbm.at[pl.ds(0, _CH)], ssem).wait()
        pltpu.make_async_copy(
            labbuf.at[0], shared_lab.at[pl.ds(0, _CH)], lsem).wait()

    def p2b(c, _):
        buf = lax.bitwise_and(c, 1)
        nbuf = lax.bitwise_and(c + 1, 1)

        @pl.when(c >= 1)
        def _():
            _wait_scatters()

        @pl.when(c + 1 < nchunks)
        def _():
            pltpu.async_copy(val_hbm.at[rlist.at[c + 1]], rows2.at[nbuf],
                             gsem)

        for s in range(_CH // _L):
            p = s * _L + lane
            r = rlist[c, pl.ds(s * _L, _L)]
            lab = plsc.load_gather(lab_v, [r])
            lab = jnp.where(c * _CH + p < cnt, lab, 0)
            plsc.store_scatter(labbuf.at[buf], [p], lab)
        _wait_gather()
        pltpu.async_copy(rows2.at[buf], out_hbm.at[my_i.at[c]], ssem)
        pltpu.async_copy(labbuf.at[buf], shared_lab.at[my_i.at[c]], lsem,
                         add=True)
        return 0

    with jax.named_scope("p2b_dma"):
        @pl.when(nchunks > 0)
        def _():
            pltpu.async_copy(val_hbm.at[rlist.at[0]], rows2.at[0], gsem)

        lax.fori_loop(0, nchunks, p2b, 0)

        @pl.when(nchunks > 0)
        def _():
            _wait_scatters()

    with jax.named_scope("p2c_labels_out"):
        plsc.subcore_barrier()
        sl = pl.ds(sid * _SL, _SL)

        @pl.when(cid == 0)
        def _():
            pltpu.sync_copy(shared_lab.at[sl], lab0_hbm.at[sl])

        @pl.when(cid == 1)
        def _():
            pltpu.sync_copy(shared_lab.at[sl], lab1_hbm.at[sl])


def kernel(idx, val, labels, buffer_imgs, buffer_labels):
    del buffer_imgs, buffer_labels
    f = pl.kernel(
        _body,
        out_type=(
            jax.ShapeDtypeStruct((_B, _DP), jnp.float32),
            jax.ShapeDtypeStruct((_B,), jnp.int32),
            jax.ShapeDtypeStruct((_B,), jnp.int32),
        ),
        mesh=plsc.VectorSubcoreMesh(core_axis_name="c", subcore_axis_name="s"),
        compiler_params=pltpu.CompilerParams(
            needs_layout_passes=False, use_tc_tiling_on_sc=False),
        scratch_types=[
            pltpu.VMEM((_B,), jnp.int32),
            pltpu.VMEM((_B,), jnp.int32),
            pltpu.VMEM((_WSZ,), jnp.int32),
            pltpu.VMEM((_NCHUNK, _CH), jnp.int32),
            pltpu.VMEM((_NCHUNK, _CH), jnp.int32),
            pltpu.VMEM((2, _CH, _DP), jnp.float32),
            pltpu.VMEM((2, _CH), jnp.int32),
            pltpu.VMEM((_B // _NS,), jnp.int32),
            pltpu.VMEM_SHARED((_B,), jnp.int32),
            pltpu.SemaphoreType.DMA,
            pltpu.SemaphoreType.DMA,
            pltpu.SemaphoreType.DMA,
            pltpu.SemaphoreType.DMA,
        ],
    )
    val_pad = jnp.pad(val, ((0, 0), (0, _DP - _D)))
    out_pad, lab0, lab1 = f(idx.astype(jnp.int32), val_pad,
                            labels.astype(jnp.int32))
    return out_pad[:, :_D], lab0 + lab1


# speedup vs baseline: 3.6330x; 1.0144x over previous
import jax
import jax.numpy as jnp
from jax import lax
from jax.experimental import pallas as pl
from jax.experimental.pallas import tpu as pltpu
from jax.experimental.pallas import tpu_sc as plsc

_B = 16384
_D = 64
_L = 16
_NC = 2
_NS = 16
_SHIFT = 15
_WSZ = 1 << _SHIFT
_DP = 128
_CH = 64
_CHS = 6
_NCHUNK = _B // _CH

_SENT = 0x7FFFFFFF


def _shift_up(x):
    i = jnp.minimum(lax.iota(jnp.int32, _L) + 1, _L - 1)
    dnums = lax.GatherDimensionNumbers(
        offset_dims=(), collapsed_slice_dims=(0,), start_index_map=(0,))
    return lax.gather(x, i[:, None], dnums, (1,),
                      mode=lax.GatherScatterMode.PROMISE_IN_BOUNDS)


def _body(idx_hbm, val_hbm, lab_hbm, out_hbm, lab0_hbm, lab1_hbm,
          idx_v, lab_v, W, my_i, rlist, rows2, labbuf, zbuf, shared_lab,
          gsem, ssem, lsem, isem):
    cid = lax.axis_index("c")
    sid = lax.axis_index("s")
    wid = sid * _NC + cid
    lane = lax.iota(jnp.int32, _L)
    _SL = _B // _NS

    _H = _B // 2
    with jax.named_scope("stage_idx"):
        cp_idx1 = pltpu.async_copy(idx_hbm.at[pl.ds(0, _H)],
                                   idx_v.at[pl.ds(0, _H)], isem)
        cp_idx2 = pltpu.async_copy(idx_hbm.at[pl.ds(_H, _H)],
                                   idx_v.at[pl.ds(_H, _H)], gsem)
        cp_lab = pltpu.async_copy(lab_hbm, lab_v, lsem)
        for s in range(_SL // _L):
            zbuf[pl.ds(s * _L, _L)] = jnp.zeros((_L,), jnp.int32)
        pltpu.sync_copy(zbuf, shared_lab.at[pl.ds(sid * _SL, _SL)])
        plsc.subcore_barrier()
        cp_idx1.wait()

    _U = 4

    def p1(kk, cnt_vec):
        sorted_keys = []
        masks = []
        for u in range(_U):
            k = kk * _U + u
            v = idx_v[pl.ds(k * _L, _L)]
            owner = lax.shift_right_logical(v, _SHIFT)
            m = owner == wid
            local = lax.bitwise_and(v, _WSZ - 1)
            key = jnp.where(m, lax.bitwise_or(lax.shift_left(local, 4), lane),
                            _SENT)
            skey, _ = plsc.sort_key_val(key, key)
            sorted_keys.append(skey)
            masks.append(m)
        for u in range(_U):
            k = kk * _U + u
            skey = sorted_keys[u]
            sloc_cmp = lax.shift_right_logical(skey, 4)
            keep = (sloc_cmp != _shift_up(sloc_cmp)) | (lane == _L - 1)
            sm = skey != _SENT
            wr = keep & sm
            sloc = lax.bitwise_and(sloc_cmp, _WSZ - 1)
            j_sorted = k * _L + lax.bitwise_and(skey, _L - 1)
            plsc.store_scatter(W, [sloc], j_sorted, mask=wr)
            addr = cnt_vec + lane
            hi = lax.shift_right_logical(addr, _CHS)
            lo = lax.bitwise_and(addr, _CH - 1)
            plsc.store_scatter(my_i, [hi, lo], j_sorted, mask=sm)
            cnt_vec = cnt_vec + plsc.all_reduce_population_count(masks[u])
        return cnt_vec

    with jax.named_scope("p1_scan"):
        half_iters = _H // (_L * _U)
        cnt_vec = lax.fori_loop(0, half_iters, p1,
                                jnp.zeros((_L,), jnp.int32))
        cp_idx2.wait()
        cnt_vec = lax.fori_loop(half_iters, 2 * half_iters, p1, cnt_vec)
        cnt = jnp.max(cnt_vec)

    nchunks = lax.shift_right_logical(cnt + _CH - 1, _CHS)
    zero16 = jnp.zeros((_L,), jnp.int32)
    i0 = plsc.load_gather(my_i, [zero16, zero16])
    l0 = lax.bitwise_and(
        plsc.load_gather(idx_v, [lax.bitwise_and(i0, _B - 1)]), _WSZ - 1)
    r0 = plsc.load_gather(W, [l0])

    def p2a(t, _):
        pos = t * _L + lane
        hi = lax.shift_right_logical(pos, _CHS)
        lo = lax.bitwise_and(pos, _CH - 1)
        valid = pos < cnt
        iv_raw = plsc.load_gather(my_i, [hi, lo])
        loc = lax.bitwise_and(
            plsc.load_gather(idx_v, [lax.bitwise_and(iv_raw, _B - 1)]),
            _WSZ - 1)
        r = jnp.where(valid, plsc.load_gather(W, [loc]), r0)
        iv = jnp.where(valid, iv_raw, i0)
        plsc.store_scatter(rlist, [hi, lo], r)
        plsc.store_scatter(my_i, [hi, lo], iv)
        return 0

    with jax.named_scope("p2a_resolve"):
        lax.fori_loop(0, nchunks * (_CH // _L), p2a, 0)
        cp_lab.wait()

    def _wait_gather():
        pltpu.make_async_copy(
            val_hbm.at[pl.ds(0, _CH)], rows2.at[0], gsem).wait()

    def _wait_scatters():
        pltpu.make_async_copy(
            rows2.at[0], out_hbm.at[pl.ds(0, _CH)], ssem).wait()
        pltpu.make_async_copy(
            labbuf.at[0], shared_lab.at[pl.ds(0, _CH)], lsem).wait()

    def p2b(c, _):
        buf = lax.bitwise_and(c, 1)
        nbuf = lax.bitwise_and(c + 1, 1)

        @pl.when(c >= 1)
        def _():
            _wait_scatters()

        @pl.when(c + 1 < nchunks)
        def _():
            pltpu.async_copy(val_hbm.at[rlist.at[c + 1]], rows2.at[nbuf],
                             gsem)

        for s in range(_CH // _L):
            p = s * _L + lane
            r = rlist[c, pl.ds(s * _L, _L)]
            lab = plsc.load_gather(lab_v, [r])
            lab = jnp.where(c * _CH + p < cnt, lab, 0)
            plsc.store_scatter(labbuf.at[buf], [p], lab)
        _wait_gather()
        pltpu.async_copy(rows2.at[buf], out_hbm.at[my_i.at[c]], ssem)
        pltpu.async_copy(labbuf.at[buf], shared_lab.at[my_i.at[c]], lsem,
                         add=True)
        return 0

    with jax.named_scope("p2b_dma"):
        @pl.when(nchunks > 0)
        def _():
            pltpu.async_copy(val_hbm.at[rlist.at[0]], rows2.at[0], gsem)

        lax.fori_loop(0, nchunks, p2b, 0)

        @pl.when(nchunks > 0)
        def _():
            _wait_scatters()

    with jax.named_scope("p2c_labels_out"):
        plsc.subcore_barrier()
        sl = pl.ds(sid * _SL, _SL)

        @pl.when(cid == 0)
        def _():
            pltpu.sync_copy(shared_lab.at[sl], lab0_hbm.at[sl])

        @pl.when(cid == 1)
        def _():
            pltpu.sync_copy(shared_lab.at[sl], lab1_hbm.at[sl])


def kernel(idx, val, labels, buffer_imgs, buffer_labels):
    del buffer_imgs, buffer_labels
    f = pl.kernel(
        _body,
        out_type=(
            jax.ShapeDtypeStruct((_B, _DP), jnp.float32),
            jax.ShapeDtypeStruct((_B,), jnp.int32),
            jax.ShapeDtypeStruct((_B,), jnp.int32),
        ),
        mesh=plsc.VectorSubcoreMesh(core_axis_name="c", subcore_axis_name="s"),
        compiler_params=pltpu.CompilerParams(
            needs_layout_passes=False, use_tc_tiling_on_sc=False),
        scratch_types=[
            pltpu.VMEM((_B,), jnp.int32),
            pltpu.VMEM((_B,), jnp.int32),
            pltpu.VMEM((_WSZ,), jnp.int32),
            pltpu.VMEM((_NCHUNK, _CH), jnp.int32),
            pltpu.VMEM((_NCHUNK, _CH), jnp.int32),
            pltpu.VMEM((2, _CH, _DP), jnp.float32),
            pltpu.VMEM((2, _CH), jnp.int32),
            pltpu.VMEM((_B // _NS,), jnp.int32),
            pltpu.VMEM_SHARED((_B,), jnp.int32),
            pltpu.SemaphoreType.DMA,
            pltpu.SemaphoreType.DMA,
            pltpu.SemaphoreType.DMA,
            pltpu.SemaphoreType.DMA,
        ],
    )
    val_pad = jnp.pad(val, ((0, 0), (0, _DP - _D)))
    out_pad, lab0, lab1 = f(idx.astype(jnp.int32), val_pad,
                            labels.astype(jnp.int32))
    return out_pad[:, :_D], lab0 + lab1


# speedup vs baseline: 3.7591x; 1.0347x over previous
import jax
import jax.numpy as jnp
from jax import lax
from jax.experimental import pallas as pl
from jax.experimental.pallas import tpu as pltpu
from jax.experimental.pallas import tpu_sc as plsc

_B = 16384
_D = 64
_L = 16
_NC = 2
_NS = 16
_SHIFT = 15
_WSZ = 1 << _SHIFT
_DP = 128
_CH = 64
_CHS = 6
_NCHUNK = _B // _CH

_SENT = 0x7FFFFFFF


def _shift_up(x):
    i = jnp.minimum(lax.iota(jnp.int32, _L) + 1, _L - 1)
    dnums = lax.GatherDimensionNumbers(
        offset_dims=(), collapsed_slice_dims=(0,), start_index_map=(0,))
    return lax.gather(x, i[:, None], dnums, (1,),
                      mode=lax.GatherScatterMode.PROMISE_IN_BOUNDS)


def _body(idx_hbm, val_hbm, lab_hbm, out_hbm, lab0_hbm, lab1_hbm,
          idx_v, lab_v, W, my_i, rlist, rows2, labbuf, zbuf, shared_lab,
          gsem, ssem, lsem, isem):
    cid = lax.axis_index("c")
    sid = lax.axis_index("s")
    wid = sid * _NC + cid
    lane = lax.iota(jnp.int32, _L)
    _SL = _B // _NS

    _H = _B // 2
    with jax.named_scope("stage_idx"):
        cp_idx1 = pltpu.async_copy(idx_hbm.at[pl.ds(0, _H)],
                                   idx_v.at[pl.ds(0, _H)], isem)
        cp_idx2 = pltpu.async_copy(idx_hbm.at[pl.ds(_H, _H)],
                                   idx_v.at[pl.ds(_H, _H)], gsem)
        cp_lab = pltpu.async_copy(lab_hbm, lab_v, lsem)
        for s in range(_SL // _L):
            zbuf[pl.ds(s * _L, _L)] = jnp.zeros((_L,), jnp.int32)
        pltpu.sync_copy(zbuf, shared_lab.at[pl.ds(sid * _SL, _SL)])
        plsc.subcore_barrier()
        cp_idx1.wait()

    _U = 8

    def p1(kk, cnt_vec):
        sorted_keys = []
        masks = []
        for u in range(_U):
            k = kk * _U + u
            v = idx_v[pl.ds(k * _L, _L)]
            owner = lax.shift_right_logical(v, _SHIFT)
            m = owner == wid
            local = lax.bitwise_and(v, _WSZ - 1)
            key = jnp.where(m, lax.bitwise_or(lax.shift_left(local, 4), lane),
                            _SENT)
            skey, _ = plsc.sort_key_val(key, key)
            sorted_keys.append(skey)
            masks.append(m)
        for u in range(_U):
            k = kk * _U + u
            skey = sorted_keys[u]
            sloc_cmp = lax.shift_right_logical(skey, 4)
            keep = (sloc_cmp != _shift_up(sloc_cmp)) | (lane == _L - 1)
            sm = skey != _SENT
            wr = keep & sm
            sloc = lax.bitwise_and(sloc_cmp, _WSZ - 1)
            j_sorted = k * _L + lax.bitwise_and(skey, _L - 1)
            plsc.store_scatter(W, [sloc], j_sorted, mask=wr)
            addr = cnt_vec + lane
            hi = lax.shift_right_logical(addr, _CHS)
            lo = lax.bitwise_and(addr, _CH - 1)
            plsc.store_scatter(my_i, [hi, lo], j_sorted, mask=sm)
            cnt_vec = cnt_vec + plsc.all_reduce_population_count(masks[u])
        return cnt_vec

    with jax.named_scope("p1_scan"):
        half_iters = _H // (_L * _U)
        cnt_vec = lax.fori_loop(0, half_iters, p1,
                                jnp.zeros((_L,), jnp.int32))
        cp_idx2.wait()
        cnt_vec = lax.fori_loop(half_iters, 2 * half_iters, p1, cnt_vec)
        cnt = jnp.max(cnt_vec)

    nchunks = lax.shift_right_logical(cnt + _CH - 1, _CHS)
    zero16 = jnp.zeros((_L,), jnp.int32)
    i0 = plsc.load_gather(my_i, [zero16, zero16])
    l0 = lax.bitwise_and(
        plsc.load_gather(idx_v, [lax.bitwise_and(i0, _B - 1)]), _WSZ - 1)
    r0 = plsc.load_gather(W, [l0])

    def p2a(t, _):
        pos = t * _L + lane
        hi = lax.shift_right_logical(pos, _CHS)
        lo = lax.bitwise_and(pos, _CH - 1)
        valid = pos < cnt
        iv_raw = plsc.load_gather(my_i, [hi, lo])
        loc = lax.bitwise_and(
            plsc.load_gather(idx_v, [lax.bitwise_and(iv_raw, _B - 1)]),
            _WSZ - 1)
        r = jnp.where(valid, plsc.load_gather(W, [loc]), r0)
        iv = jnp.where(valid, iv_raw, i0)
        plsc.store_scatter(rlist, [hi, lo], r)
        plsc.store_scatter(my_i, [hi, lo], iv)
        return 0

    with jax.named_scope("p2a_resolve"):
        lax.fori_loop(0, nchunks * (_CH // _L), p2a, 0)
        cp_lab.wait()

    def _wait_gather():
        pltpu.make_async_copy(
            val_hbm.at[pl.ds(0, _CH)], rows2.at[0], gsem).wait()

    def _wait_scatters():
        pltpu.make_async_copy(
            rows2.at[0], out_hbm.at[pl.ds(0, _CH)], ssem).wait()
        pltpu.make_async_copy(
            labbuf.at[0], shared_lab.at[pl.ds(0, _CH)], lsem).wait()

    def p2b(c, _):
        buf = lax.bitwise_and(c, 1)
        nbuf = lax.bitwise_and(c + 1, 1)

        @pl.when(c >= 1)
        def _():
            _wait_scatters()

        @pl.when(c + 1 < nchunks)
        def _():
            pltpu.async_copy(val_hbm.at[rlist.at[c + 1]], rows2.at[nbuf],
                             gsem)

        for s in range(_CH // _L):
            p = s * _L + lane
            r = rlist[c, pl.ds(s * _L, _L)]
            lab = plsc.load_gather(lab_v, [r])
            lab = jnp.where(c * _CH + p < cnt, lab, 0)
            plsc.store_scatter(labbuf.at[buf], [p], lab)
        _wait_gather()
        pltpu.async_copy(rows2.at[buf], out_hbm.at[my_i.at[c]], ssem)
        pltpu.async_copy(labbuf.at[buf], shared_lab.at[my_i.at[c]], lsem,
                         add=True)
        return 0

    with jax.named_scope("p2b_dma"):
        @pl.when(nchunks > 0)
        def _():
            pltpu.async_copy(val_hbm.at[rlist.at[0]], rows2.at[0], gsem)

        lax.fori_loop(0, nchunks, p2b, 0)

        @pl.when(nchunks > 0)
        def _():
            _wait_scatters()

    with jax.named_scope("p2c_labels_out"):
        plsc.subcore_barrier()
        sl = pl.ds(sid * _SL, _SL)

        @pl.when(cid == 0)
        def _():
            pltpu.sync_copy(shared_lab.at[sl], lab0_hbm.at[sl])

        @pl.when(cid == 1)
        def _():
            pltpu.sync_copy(shared_lab.at[sl], lab1_hbm.at[sl])


def kernel(idx, val, labels, buffer_imgs, buffer_labels):
    del buffer_imgs, buffer_labels
    f = pl.kernel(
        _body,
        out_type=(
            jax.ShapeDtypeStruct((_B, _DP), jnp.float32),
            jax.ShapeDtypeStruct((_B,), jnp.int32),
            jax.ShapeDtypeStruct((_B,), jnp.int32),
        ),
        mesh=plsc.VectorSubcoreMesh(core_axis_name="c", subcore_axis_name="s"),
        compiler_params=pltpu.CompilerParams(
            needs_layout_passes=False, use_tc_tiling_on_sc=False),
        scratch_types=[
            pltpu.VMEM((_B,), jnp.int32),
            pltpu.VMEM((_B,), jnp.int32),
            pltpu.VMEM((_WSZ,), jnp.int32),
            pltpu.VMEM((_NCHUNK, _CH), jnp.int32),
            pltpu.VMEM((_NCHUNK, _CH), jnp.int32),
            pltpu.VMEM((2, _CH, _DP), jnp.float32),
            pltpu.VMEM((2, _CH), jnp.int32),
            pltpu.VMEM((_B // _NS,), jnp.int32),
            pltpu.VMEM_SHARED((_B,), jnp.int32),
            pltpu.SemaphoreType.DMA,
            pltpu.SemaphoreType.DMA,
            pltpu.SemaphoreType.DMA,
            pltpu.SemaphoreType.DMA,
        ],
    )
    val_pad = jnp.pad(val, ((0, 0), (0, _DP - _D)))
    out_pad, lab0, lab1 = f(idx.astype(jnp.int32), val_pad,
                            labels.astype(jnp.int32))
    return out_pad[:, :_D], lab0 + lab1
